# baseline (speedup 1.0000x reference)
import functools

import jax
import jax.numpy as jnp
from jax.experimental import pallas as pl
from jax.experimental.pallas import tpu as pltpu

_BLOCK = 1024


def _fused_mlp_kernel(x_ref, kt_ref, a0c_ref, a0s_ref, b0_ref, a1_ref, b1_ref,
                      a2_ref, b2_ref, a3_ref, b3_ref, a4_ref, b4_ref, y_ref):
    x = x_ref[...]
    kt = kt_ref[...]
    z = (x[:, 0:1] * kt[0:1, :]
         + x[:, 1:2] * kt[1:2, :]
         + x[:, 2:3] * kt[2:3, :])
    h = jnp.tanh(jnp.cos(z) @ a0c_ref[...] + jnp.sin(z) @ a0s_ref[...]
                 + b0_ref[...])
    h = jnp.tanh(h @ a1_ref[...] + b1_ref[...])
    h = jnp.tanh(h @ a2_ref[...] + b2_ref[...])
    h = jnp.tanh(h @ a3_ref[...] + b3_ref[...])
    y_ref[...] = h @ a4_ref[...] + b4_ref[...]


@jax.jit
def kernel(x, kernel_rff, W0, b0, W1, b1, W2, b2, W3, b3, W4, b4):
    n = x.shape[0]
    d0 = W0.shape[1]
    half = d0 // 2
    kt = kernel_rff.T
    a0 = W0.T
    a0c, a0s = a0[:half], a0[half:]
    a1, a2, a3, a4 = W1.T, W2.T, W3.T, W4.T
    grid = (n // _BLOCK,)

    def rows(i):
        return (i, 0)

    def whole(i):
        return (0, 0)

    full = lambda arr: pl.BlockSpec(arr.shape, whole)
    out = pl.pallas_call(
        _fused_mlp_kernel,
        grid=grid,
        in_specs=[
            pl.BlockSpec((_BLOCK, 3), rows),
            full(kt),
            full(a0c), full(a0s), pl.BlockSpec((1, b0.shape[0]), whole),
            full(a1), pl.BlockSpec((1, b1.shape[0]), whole),
            full(a2), pl.BlockSpec((1, b2.shape[0]), whole),
            full(a3), pl.BlockSpec((1, b3.shape[0]), whole),
            full(a4), pl.BlockSpec((1, b4.shape[0]), whole),
        ],
        out_specs=pl.BlockSpec((_BLOCK, 3), rows),
        out_shape=jax.ShapeDtypeStruct((n, 3), x.dtype),
        compiler_params=pltpu.CompilerParams(
            dimension_semantics=("parallel",),
        ),
    )(x, kt, a0c, a0s, b0[None, :], a1, b1[None, :], a2, b2[None, :],
      a3, b3[None, :], a4, b4[None, :])
    return out
